# baseline (speedup 1.0000x reference)
import jax
import jax.numpy as jnp
from jax import lax
from jax.experimental import pallas as pl
from jax.experimental.pallas import tpu as pltpu
from jax.experimental.pallas import tpu_sc as plsc

BN = 2048
NP = 100352
NC = 2
NS = 16
NW = NC * NS
B_PER_W = NP // NW
KCH = 392
NCH = B_PER_W // KCH


def _sc_gather_body(x_hbm, e_hbm, g_hbm, idx_v, r_v, gsem, wsem):
    wid = lax.axis_index("c") * NS + lax.axis_index("s")
    base = wid * B_PER_W
    pltpu.sync_copy(e_hbm.at[pl.ds(base, B_PER_W)], idx_v)

    def start_gather(j):
        return pltpu.async_copy(
            x_hbm.at[idx_v.at[pl.ds(j * KCH, KCH)]], r_v.at[j % 2], gsem)

    def start_writeback(j):
        return pltpu.async_copy(
            r_v.at[j % 2], g_hbm.at[pl.ds(base + j * KCH, KCH)], wsem)

    g_cp = {0: start_gather(0)}
    w_cp = {}
    for j in range(NCH):
        if j + 1 < NCH:
            if j - 1 in w_cp:
                w_cp.pop(j - 1).wait()
            g_cp[j + 1] = start_gather(j + 1)
        g_cp.pop(j).wait()
        w_cp[j] = start_writeback(j)
    for j in sorted(w_cp):
        w_cp[j].wait()


def _combine_body(x_ref, g_ref, ws_ref, wn_ref, vm_ref, out_ref):
    xb = x_ref[...]
    g = g_ref[...]
    lane = lax.broadcasted_iota(jnp.int32, xb.shape, 1)
    first3 = lane < 3
    comb = jnp.where(first3, g, g + xb)
    diff = xb - g
    sel3 = jnp.where(lane[0:1, :] < 3, jnp.float32(1.0), jnp.float32(0.0))
    dt = (((1,), (1,)), ((), ()))
    d2 = lax.dot_general(sel3, diff * diff, dt,
                         preferred_element_type=jnp.float32)
    rec = vm_ref[...] / jnp.where(d2 > 0.0, d2, jnp.float32(1e-4))
    dn = (((0,), (1,)), ((), ()))
    r = lax.dot_general(wn_ref[...], comb, dn,
                        preferred_element_type=jnp.float32)
    s = lax.dot_general(ws_ref[...], xb, dn,
                        preferred_element_type=jnp.float32)
    out_ref[...] = r * rec + s * vm_ref[...]


def kernel(x, neighbor_idx, w_s, w_n):
    n, f = x.shape
    c = w_s.shape[1]
    grid = NP // BN

    idx0 = neighbor_idx[:, 0]
    idx1 = neighbor_idx[:, 1]
    e = jnp.where(idx1 != 0, idx1, idx0)
    e_pad = jnp.concatenate([e, jnp.zeros((NP - n,), jnp.int32)])
    vmask = (e != 0).astype(jnp.float32).reshape(1, n)

    mesh = plsc.VectorSubcoreMesh(core_axis_name="c", subcore_axis_name="s",
                                  num_cores=NC, num_subcores=NS)
    g = pl.kernel(
        _sc_gather_body,
        out_type=jax.ShapeDtypeStruct((NP, f), jnp.float32),
        mesh=mesh,
        scratch_types=[
            pltpu.VMEM((B_PER_W,), jnp.int32),
            pltpu.VMEM((2, KCH, f), jnp.float32),
            pltpu.SemaphoreType.DMA,
            pltpu.SemaphoreType.DMA,
        ],
    )(x, e_pad)

    out_t = pl.pallas_call(
        _combine_body,
        grid=(grid,),
        in_specs=[
            pl.BlockSpec((BN, f), lambda i: (i, 0)),
            pl.BlockSpec((BN, f), lambda i: (i, 0)),
            pl.BlockSpec((f, c), lambda i: (0, 0)),
            pl.BlockSpec((f, c), lambda i: (0, 0)),
            pl.BlockSpec((1, BN), lambda i: (0, i)),
        ],
        out_specs=pl.BlockSpec((c, BN), lambda i: (0, i)),
        out_shape=jax.ShapeDtypeStruct((c, n), jnp.float32),
    )(x, g, w_s, w_n, vmask)

    return out_t.T


# speedup vs baseline: 1.0120x; 1.0120x over previous
import jax
import jax.numpy as jnp
from jax import lax
from jax.experimental import pallas as pl
from jax.experimental.pallas import tpu as pltpu
from jax.experimental.pallas import tpu_sc as plsc

BN = 2048
NP = 100352
NC = 2
NS = 16
NW = NC * NS
NCH = 8
KCH0 = 472
KCH1 = 312
R0 = NCH * KCH0
R1 = NCH * KCH1


def _sc_gather_body(x_hbm, e_hbm, g_hbm, idx_v, r_v, gsem, wsem):
    cc = lax.axis_index("c")
    ss = lax.axis_index("s")

    def pipeline(base, kch):
        rows = NCH * kch
        pltpu.sync_copy(e_hbm.at[pl.ds(base, rows)], idx_v.at[pl.ds(0, rows)])

        def start_gather(j):
            return pltpu.async_copy(
                x_hbm.at[idx_v.at[pl.ds(j * kch, kch)]],
                r_v.at[j % 2, pl.ds(0, kch)], gsem)

        def start_writeback(j):
            return pltpu.async_copy(
                r_v.at[j % 2, pl.ds(0, kch)],
                g_hbm.at[pl.ds(base + j * kch, kch)], wsem)

        g_cp = {0: start_gather(0)}
        w_cp = {}
        for j in range(NCH):
            if j + 1 < NCH:
                if j - 1 in w_cp:
                    w_cp.pop(j - 1).wait()
                g_cp[j + 1] = start_gather(j + 1)
            g_cp.pop(j).wait()
            w_cp[j] = start_writeback(j)
        for j in sorted(w_cp):
            w_cp[j].wait()

    @pl.when(cc == 0)
    def _():
        pipeline(ss * R0, KCH0)

    @pl.when(cc == 1)
    def _():
        pipeline(NS * R0 + ss * R1, KCH1)


def _combine_body(x_ref, g_ref, ws_ref, wn_ref, vm_ref, out_ref):
    xb = x_ref[...]
    g = g_ref[...]
    lane = lax.broadcasted_iota(jnp.int32, xb.shape, 1)
    first3 = lane < 3
    comb = jnp.where(first3, g, g + xb)
    diff = xb - g
    sel3 = jnp.where(lane[0:1, :] < 3, jnp.float32(1.0), jnp.float32(0.0))
    dt = (((1,), (1,)), ((), ()))
    d2 = lax.dot_general(sel3, diff * diff, dt,
                         preferred_element_type=jnp.float32)
    rec = vm_ref[...] / jnp.where(d2 > 0.0, d2, jnp.float32(1e-4))
    dn = (((0,), (1,)), ((), ()))
    r = lax.dot_general(wn_ref[...], comb, dn,
                        preferred_element_type=jnp.float32)
    s = lax.dot_general(ws_ref[...], xb, dn,
                        preferred_element_type=jnp.float32)
    out_ref[...] = r * rec + s * vm_ref[...]


def kernel(x, neighbor_idx, w_s, w_n):
    n, f = x.shape
    c = w_s.shape[1]
    grid = NP // BN

    idx0 = neighbor_idx[:, 0]
    idx1 = neighbor_idx[:, 1]
    e = jnp.where(idx1 != 0, idx1, idx0)
    e_pad = jnp.concatenate([e, jnp.zeros((NP - n,), jnp.int32)])
    vmask = (e != 0).astype(jnp.float32).reshape(1, n)

    mesh = plsc.VectorSubcoreMesh(core_axis_name="c", subcore_axis_name="s",
                                  num_cores=NC, num_subcores=NS)
    g = pl.kernel(
        _sc_gather_body,
        out_type=jax.ShapeDtypeStruct((NP, f), jnp.float32),
        mesh=mesh,
        scratch_types=[
            pltpu.VMEM((R0,), jnp.int32),
            pltpu.VMEM((2, KCH0, f), jnp.float32),
            pltpu.SemaphoreType.DMA,
            pltpu.SemaphoreType.DMA,
        ],
    )(x, e_pad)

    out_t = pl.pallas_call(
        _combine_body,
        grid=(grid,),
        in_specs=[
            pl.BlockSpec((BN, f), lambda i: (i, 0)),
            pl.BlockSpec((BN, f), lambda i: (i, 0)),
            pl.BlockSpec((f, c), lambda i: (0, 0)),
            pl.BlockSpec((f, c), lambda i: (0, 0)),
            pl.BlockSpec((1, BN), lambda i: (0, i)),
        ],
        out_specs=pl.BlockSpec((c, BN), lambda i: (0, i)),
        out_shape=jax.ShapeDtypeStruct((c, n), jnp.float32),
    )(x, g, w_s, w_n, vmask)

    return out_t.T


# speedup vs baseline: 1.1132x; 1.1000x over previous
import jax
import jax.numpy as jnp
from jax import lax
from jax.experimental import pallas as pl
from jax.experimental.pallas import tpu as pltpu
from jax.experimental.pallas import tpu_sc as plsc

BN = 3584
NP = 100352
NC = 2
NS = 16
NW = NC * NS
NCH = 8
KCH0 = 472
KCH1 = 312
R0 = NCH * KCH0
R1 = NCH * KCH1


def _sc_gather_body(x_hbm, e_hbm, g_hbm, idx_v, r_v, gsem, wsem):
    cc = lax.axis_index("c")
    ss = lax.axis_index("s")

    def pipeline(base, kch):
        rows = NCH * kch
        pltpu.sync_copy(e_hbm.at[pl.ds(base, rows)], idx_v.at[pl.ds(0, rows)])

        def start_gather(j):
            return pltpu.async_copy(
                x_hbm.at[idx_v.at[pl.ds(j * kch, kch)]],
                r_v.at[j % 2, pl.ds(0, kch)], gsem)

        def start_writeback(j):
            return pltpu.async_copy(
                r_v.at[j % 2, pl.ds(0, kch)],
                g_hbm.at[pl.ds(base + j * kch, kch)], wsem)

        g_cp = {0: start_gather(0)}
        w_cp = {}
        for j in range(NCH):
            if j + 1 < NCH:
                if j - 1 in w_cp:
                    w_cp.pop(j - 1).wait()
                g_cp[j + 1] = start_gather(j + 1)
            g_cp.pop(j).wait()
            w_cp[j] = start_writeback(j)
        for j in sorted(w_cp):
            w_cp[j].wait()

    @pl.when(cc == 0)
    def _():
        pipeline(ss * R0, KCH0)

    @pl.when(cc == 1)
    def _():
        pipeline(NS * R0 + ss * R1, KCH1)


def _combine_body(x_ref, g_ref, ws_ref, wn_ref, vm_ref, out_ref):
    xb = x_ref[...]
    g = g_ref[...]
    lane = lax.broadcasted_iota(jnp.int32, xb.shape, 1)
    first3 = lane < 3
    comb = jnp.where(first3, g, g + xb)
    diff = xb - g
    sel3 = jnp.where(lane[0:1, :] < 3, jnp.float32(1.0), jnp.float32(0.0))
    dt = (((1,), (1,)), ((), ()))
    d2 = lax.dot_general(sel3, diff * diff, dt,
                         preferred_element_type=jnp.float32)
    rec = vm_ref[...] / jnp.where(d2 > 0.0, d2, jnp.float32(1e-4))
    dn = (((0,), (1,)), ((), ()))
    r = lax.dot_general(wn_ref[...], comb, dn,
                        preferred_element_type=jnp.float32)
    s = lax.dot_general(ws_ref[...], xb, dn,
                        preferred_element_type=jnp.float32)
    out_ref[...] = r * rec + s * vm_ref[...]


def kernel(x, neighbor_idx, w_s, w_n):
    n, f = x.shape
    c = w_s.shape[1]
    grid = NP // BN

    idx0 = neighbor_idx[:, 0]
    idx1 = neighbor_idx[:, 1]
    e = jnp.where(idx1 != 0, idx1, idx0)
    e_pad = jnp.concatenate([e, jnp.zeros((NP - n,), jnp.int32)])
    vmask = (e != 0).astype(jnp.float32).reshape(1, n)

    mesh = plsc.VectorSubcoreMesh(core_axis_name="c", subcore_axis_name="s",
                                  num_cores=NC, num_subcores=NS)
    g = pl.kernel(
        _sc_gather_body,
        out_type=jax.ShapeDtypeStruct((NP, f), jnp.float32),
        mesh=mesh,
        scratch_types=[
            pltpu.VMEM((R0,), jnp.int32),
            pltpu.VMEM((2, KCH0, f), jnp.float32),
            pltpu.SemaphoreType.DMA,
            pltpu.SemaphoreType.DMA,
        ],
    )(x, e_pad)

    out_t = pl.pallas_call(
        _combine_body,
        grid=(grid,),
        in_specs=[
            pl.BlockSpec((BN, f), lambda i: (i, 0)),
            pl.BlockSpec((BN, f), lambda i: (i, 0)),
            pl.BlockSpec((f, c), lambda i: (0, 0)),
            pl.BlockSpec((f, c), lambda i: (0, 0)),
            pl.BlockSpec((1, BN), lambda i: (0, i)),
        ],
        out_specs=pl.BlockSpec((c, BN), lambda i: (0, i)),
        out_shape=jax.ShapeDtypeStruct((c, n), jnp.float32),
    )(x, g, w_s, w_n, vmask)

    return out_t.T


# speedup vs baseline: 1.1791x; 1.0592x over previous
import jax
import jax.numpy as jnp
from jax import lax
from jax.experimental import pallas as pl
from jax.experimental.pallas import tpu as pltpu
from jax.experimental.pallas import tpu_sc as plsc

BN = 7168
NP = 100352
NC = 2
NS = 16
NW = NC * NS
NCH = 8
KCH0 = 472
KCH1 = 312
R0 = NCH * KCH0
R1 = NCH * KCH1


def _sc_gather_body(x_hbm, e_hbm, g_hbm, idx_v, r_v, gsem, wsem):
    cc = lax.axis_index("c")
    ss = lax.axis_index("s")

    def pipeline(base, kch):
        rows = NCH * kch
        pltpu.sync_copy(e_hbm.at[pl.ds(base, rows)], idx_v.at[pl.ds(0, rows)])

        def start_gather(j):
            return pltpu.async_copy(
                x_hbm.at[idx_v.at[pl.ds(j * kch, kch)]],
                r_v.at[j % 2, pl.ds(0, kch)], gsem)

        def start_writeback(j):
            return pltpu.async_copy(
                r_v.at[j % 2, pl.ds(0, kch)],
                g_hbm.at[pl.ds(base + j * kch, kch)], wsem)

        g_cp = {0: start_gather(0)}
        w_cp = {}
        for j in range(NCH):
            if j + 1 < NCH:
                if j - 1 in w_cp:
                    w_cp.pop(j - 1).wait()
                g_cp[j + 1] = start_gather(j + 1)
            g_cp.pop(j).wait()
            w_cp[j] = start_writeback(j)
        for j in sorted(w_cp):
            w_cp[j].wait()

    @pl.when(cc == 0)
    def _():
        pipeline(ss * R0, KCH0)

    @pl.when(cc == 1)
    def _():
        pipeline(NS * R0 + ss * R1, KCH1)


def _combine_body(x_ref, g_ref, ws_ref, wn_ref, vm_ref, out_ref):
    xb = x_ref[...]
    g = g_ref[...]
    lane = lax.broadcasted_iota(jnp.int32, xb.shape, 1)
    first3 = lane < 3
    comb = jnp.where(first3, g, g + xb)
    diff = xb - g
    sel3 = jnp.where(lane[0:1, :] < 3, jnp.float32(1.0), jnp.float32(0.0))
    dt = (((1,), (1,)), ((), ()))
    d2 = lax.dot_general(sel3, diff * diff, dt,
                         preferred_element_type=jnp.float32)
    rec = vm_ref[...] / jnp.where(d2 > 0.0, d2, jnp.float32(1e-4))
    dn = (((0,), (1,)), ((), ()))
    r = lax.dot_general(wn_ref[...], comb, dn,
                        preferred_element_type=jnp.float32)
    s = lax.dot_general(ws_ref[...], xb, dn,
                        preferred_element_type=jnp.float32)
    out_ref[...] = r * rec + s * vm_ref[...]


def kernel(x, neighbor_idx, w_s, w_n):
    n, f = x.shape
    c = w_s.shape[1]
    grid = NP // BN

    idx0 = neighbor_idx[:, 0]
    idx1 = neighbor_idx[:, 1]
    e = jnp.where(idx1 != 0, idx1, idx0)
    e_pad = jnp.concatenate([e, jnp.zeros((NP - n,), jnp.int32)])
    vmask = (e != 0).astype(jnp.float32).reshape(1, n)

    mesh = plsc.VectorSubcoreMesh(core_axis_name="c", subcore_axis_name="s",
                                  num_cores=NC, num_subcores=NS)
    g = pl.kernel(
        _sc_gather_body,
        out_type=jax.ShapeDtypeStruct((NP, f), jnp.float32),
        mesh=mesh,
        scratch_types=[
            pltpu.VMEM((R0,), jnp.int32),
            pltpu.VMEM((2, KCH0, f), jnp.float32),
            pltpu.SemaphoreType.DMA,
            pltpu.SemaphoreType.DMA,
        ],
    )(x, e_pad)

    out_t = pl.pallas_call(
        _combine_body,
        grid=(grid,),
        in_specs=[
            pl.BlockSpec((BN, f), lambda i: (i, 0)),
            pl.BlockSpec((BN, f), lambda i: (i, 0)),
            pl.BlockSpec((f, c), lambda i: (0, 0)),
            pl.BlockSpec((f, c), lambda i: (0, 0)),
            pl.BlockSpec((1, BN), lambda i: (0, i)),
        ],
        out_specs=pl.BlockSpec((c, BN), lambda i: (0, i)),
        out_shape=jax.ShapeDtypeStruct((c, n), jnp.float32),
    )(x, g, w_s, w_n, vmask)

    return out_t.T
